# baseline (speedup 1.0000x reference)
import functools

import jax
import jax.numpy as jnp
from jax import lax
from jax.experimental import pallas as pl
from jax.experimental.pallas import tpu as pltpu
from jax.experimental.pallas import tpu_sc as plsc

_NC, _NS = 2, 16
_NW = _NC * _NS


def _proj_body(x_ref, w_ref, o_ref):
    o_ref[...] = jnp.dot(x_ref[...], w_ref[...],
                         preferred_element_type=jnp.float32)


def _project(x, w, blk=1000, interpret=False):
    n, d = x.shape
    dout = w.shape[1]
    return pl.pallas_call(
        _proj_body,
        grid=(n // blk,),
        in_specs=[
            pl.BlockSpec((blk, d), lambda i: (i, 0)),
            pl.BlockSpec((d, dout), lambda i: (0, 0)),
        ],
        out_specs=pl.BlockSpec((blk, dout), lambda i: (i, 0)),
        out_shape=jax.ShapeDtypeStruct((n, dout), jnp.float32),
        interpret=interpret,
    )(x, w)


_NBUF = 2


def _sc_gather(table, idx2d):
    n_rows, ch = idx2d.shape
    n_table, d = table.shape
    rows_per_w = n_rows // _NW
    n_rounds = rows_per_w // _NBUF
    e_pad = n_rows * ch
    n_fill = 10
    fill_rows = n_table // n_fill
    mesh = plsc.VectorSubcoreMesh(core_axis_name="c", subcore_axis_name="s")

    @functools.partial(
        pl.kernel,
        mesh=mesh,
        out_type=jax.ShapeDtypeStruct((e_pad, d), jnp.float32),
        scratch_types=[
            pltpu.VMEM((rows_per_w, ch), jnp.int32),
            pltpu.VMEM((_NBUF, ch, d), jnp.float32),
            pltpu.VMEM_SHARED((n_table, d), jnp.float32),
        ]
        + [pltpu.SemaphoreType.DMA] * (2 * _NBUF),
    )
    def gk(table_hbm, idx_hbm, out_hbm, idx_v, rows_v, shared, *sems):
        gs, ws = sems[:_NBUF], sems[_NBUF:]
        s_ax = lax.axis_index("s")
        wid = s_ax * _NC + lax.axis_index("c")
        idx_base = wid * rows_per_w
        out_base = idx_base * ch

        @pl.when(s_ax < n_fill)
        def _():
            pltpu.sync_copy(
                table_hbm.at[pl.ds(s_ax * fill_rows, fill_rows)],
                shared.at[pl.ds(s_ax * fill_rows, fill_rows)])

        pltpu.sync_copy(idx_hbm.at[pl.ds(idx_base, rows_per_w)], idx_v)
        plsc.subcore_barrier()

        for b in range(_NBUF):
            pltpu.async_copy(shared.at[idx_v.at[b]], rows_v.at[b], gs[b])

        def body(j, carry):
            base_c = j * _NBUF
            for b in range(_NBUF):
                pltpu.make_async_copy(
                    out_hbm.at[pl.ds(0, ch)], rows_v.at[b], gs[b]).wait()
                pltpu.async_copy(
                    rows_v.at[b],
                    out_hbm.at[pl.ds(out_base + (base_c + b) * ch, ch)],
                    ws[b])

            @pl.when(j < n_rounds - 1)
            def _():
                for b in range(_NBUF):
                    pltpu.make_async_copy(
                        rows_v.at[b], out_hbm.at[pl.ds(0, ch)], ws[b]).wait()
                    pltpu.async_copy(
                        shared.at[idx_v.at[base_c + _NBUF + b]],
                        rows_v.at[b], gs[b])

            return carry

        lax.fori_loop(0, n_rounds, body, 0)
        for b in range(_NBUF):
            pltpu.make_async_copy(
                rows_v.at[b], out_hbm.at[pl.ds(0, ch)], ws[b]).wait()

    return gk(table, idx2d)


def _edge_node_body(nodes_ref, edges_ref, gath_ref, wa_ref, wc_ref, we2_ref,
                    wn1a_ref, wn1b_ref, wn2_ref, be1_ref, be2_ref, bn1_ref,
                    bn2_ref, nn_ref, ne_ref):
    p, nl = nodes_ref.shape
    nv = edges_ref.shape[0] // p
    n = nodes_ref[...]
    t = jnp.dot(edges_ref[...], wc_ref[...].astype(jnp.bfloat16),
                preferred_element_type=jnp.float32)
    selfp = jnp.dot(n, wa_ref[...], preferred_element_type=jnp.float32)
    s3 = jnp.broadcast_to(selfp[:, None, :], (p, nv, nl)).reshape(p * nv, nl)
    h = jnp.maximum(t + gath_ref[...] + s3 + be1_ref[...], 0.0)
    ne = jnp.dot(h.astype(jnp.bfloat16), we2_ref[...].astype(jnp.bfloat16),
                 preferred_element_type=jnp.float32) + be2_ref[...]
    ne_ref[...] = ne
    agg = jnp.sum(ne.reshape(p, nv, nl), axis=1)
    h2 = jnp.maximum(
        jnp.dot(n, wn1a_ref[...], preferred_element_type=jnp.float32)
        + jnp.dot(agg, wn1b_ref[...], preferred_element_type=jnp.float32)
        + bn1_ref[...], 0.0)
    nn_ref[...] = n + jnp.dot(h2, wn2_ref[...],
                              preferred_element_type=jnp.float32) + bn2_ref[...]


def _edge_node(nodes2d, edges2d, gathered, wa, wc, we2, wn1a, wn1b, wn2,
               be1, be2, bn1, bn2, blk=400, interpret=False):
    n_p, nl = nodes2d.shape
    e = edges2d.shape[0]
    nv = e // n_p
    grid = (n_p // blk,)
    wspec = pl.BlockSpec((nl, nl), lambda i: (0, 0))
    bspec = pl.BlockSpec((1, nl), lambda i: (0, 0))
    return pl.pallas_call(
        _edge_node_body,
        grid=grid,
        in_specs=[
            pl.BlockSpec((blk, nl), lambda i: (i, 0)),
            pl.BlockSpec((blk * nv, nl), lambda i: (i, 0)),
            pl.BlockSpec((blk * nv, nl), lambda i: (i, 0)),
            wspec, wspec, wspec, wspec, wspec, wspec,
            bspec, bspec, bspec, bspec,
        ],
        out_specs=[
            pl.BlockSpec((blk, nl), lambda i: (i, 0)),
            pl.BlockSpec((blk * nv, nl), lambda i: (i, 0)),
        ],
        out_shape=[
            jax.ShapeDtypeStruct((n_p, nl), jnp.float32),
            jax.ShapeDtypeStruct((e, nl), jnp.float32),
        ],
        interpret=interpret,
    )(nodes2d, edges2d, gathered, wa, wc, we2, wn1a, wn1b, wn2,
      be1, be2, bn1, bn2)


def kernel(nodes, edges_attributes, edges_indices, We1, be1, We2, be2,
           Wn1, bn1, Wn2, bn2):
    b, n_p, nl = nodes.shape
    nv = edges_attributes.shape[1]
    e = n_p * nv

    nodes2d = nodes.reshape(n_p, nl)
    edges2d = edges_attributes.reshape(e, nl).astype(jnp.bfloat16)
    wa, wb, wc = We1[:nl], We1[nl:2 * nl], We1[2 * nl:]
    wn1a, wn1b = Wn1[:nl], Wn1[nl:]
    be1_2 = be1.reshape(1, nl)
    be2_2 = be2.reshape(1, nl)
    bn1_2 = bn1.reshape(1, nl)
    bn2_2 = bn2.reshape(1, nl)

    neighproj = _project(nodes2d, wb)

    ch = 128
    chunk = _NW * ch
    e_pad = ((e + chunk - 1) // chunk) * chunk
    flat_idx = edges_indices.reshape(e).astype(jnp.int32)
    flat_idx = jnp.pad(flat_idx, (0, e_pad - e))
    idx2d = flat_idx.reshape(e_pad // ch, ch)

    gathered = _sc_gather(neighproj, idx2d)

    new_nodes2d, new_edges2d = _edge_node(
        nodes2d, edges2d, gathered, wa, wc, We2, wn1a, wn1b, Wn2,
        be1_2, be2_2, bn1_2, bn2_2)

    return (new_nodes2d.reshape(b, n_p, nl),
            new_edges2d.reshape(n_p, nv, nl))


# speedup vs baseline: 1.0761x; 1.0761x over previous
import functools

import jax
import jax.numpy as jnp
from jax import lax
from jax.experimental import pallas as pl
from jax.experimental.pallas import tpu as pltpu
from jax.experimental.pallas import tpu_sc as plsc

_NC, _NS = 2, 16
_NW = _NC * _NS


def _proj_body(x_ref, w_ref, o_ref):
    o_ref[...] = jnp.dot(x_ref[...], w_ref[...],
                         preferred_element_type=jnp.float32)


def _project(x, w, blk=1000, interpret=False):
    n, d = x.shape
    dout = w.shape[1]
    return pl.pallas_call(
        _proj_body,
        grid=(n // blk,),
        in_specs=[
            pl.BlockSpec((blk, d), lambda i: (i, 0)),
            pl.BlockSpec((d, dout), lambda i: (0, 0)),
        ],
        out_specs=pl.BlockSpec((blk, dout), lambda i: (i, 0)),
        out_shape=jax.ShapeDtypeStruct((n, dout), jnp.float32),
        interpret=interpret,
    )(x, w)


_NBUF = 2


def _sc_gather(table, idx2d):
    n_rows, ch = idx2d.shape
    n_table, d = table.shape
    rows_per_w = n_rows // _NW
    n_rounds = rows_per_w // _NBUF
    e_pad = n_rows * ch
    n_fill = 10
    fill_rows = n_table // n_fill
    mesh = plsc.VectorSubcoreMesh(core_axis_name="c", subcore_axis_name="s")

    @functools.partial(
        pl.kernel,
        mesh=mesh,
        out_type=jax.ShapeDtypeStruct((e_pad, d), jnp.float32),
        scratch_types=[
            pltpu.VMEM((rows_per_w, ch), jnp.int32),
            pltpu.VMEM((_NBUF, ch, d), jnp.float32),
            pltpu.VMEM_SHARED((n_table, d), jnp.float32),
        ]
        + [pltpu.SemaphoreType.DMA] * (2 * _NBUF),
    )
    def gk(table_hbm, idx_hbm, out_hbm, idx_v, rows_v, shared, *sems):
        gs, ws = sems[:_NBUF], sems[_NBUF:]
        s_ax = lax.axis_index("s")
        wid = s_ax * _NC + lax.axis_index("c")
        idx_base = wid * rows_per_w
        out_base = idx_base * ch

        @pl.when(s_ax < n_fill)
        def _():
            pltpu.sync_copy(
                table_hbm.at[pl.ds(s_ax * fill_rows, fill_rows)],
                shared.at[pl.ds(s_ax * fill_rows, fill_rows)])

        pltpu.sync_copy(idx_hbm.at[pl.ds(idx_base, rows_per_w)], idx_v)
        plsc.subcore_barrier()

        for b in range(_NBUF):
            pltpu.async_copy(shared.at[idx_v.at[b]], rows_v.at[b], gs[b])

        def body(j, carry):
            base_c = j * _NBUF
            for b in range(_NBUF):
                pltpu.make_async_copy(
                    out_hbm.at[pl.ds(0, ch)], rows_v.at[b], gs[b]).wait()
                pltpu.async_copy(
                    rows_v.at[b],
                    out_hbm.at[pl.ds(out_base + (base_c + b) * ch, ch)],
                    ws[b])

            @pl.when(j < n_rounds - 1)
            def _():
                for b in range(_NBUF):
                    pltpu.make_async_copy(
                        rows_v.at[b], out_hbm.at[pl.ds(0, ch)], ws[b]).wait()
                    pltpu.async_copy(
                        shared.at[idx_v.at[base_c + _NBUF + b]],
                        rows_v.at[b], gs[b])

            return carry

        lax.fori_loop(0, n_rounds, body, 0)
        for b in range(_NBUF):
            pltpu.make_async_copy(
                rows_v.at[b], out_hbm.at[pl.ds(0, ch)], ws[b]).wait()

    return gk(table, idx2d)


def _edge_node_body(nodes_ref, edges_ref, gath_ref, wa_ref, wc_ref, we2_ref,
                    wn1a_ref, wn1b_ref, wn2_ref, be1_ref, be2_ref, bn1_ref,
                    bn2_ref, nn_ref, ne_ref):
    p, nl = nodes_ref.shape
    nv = edges_ref.shape[0] // p
    n = nodes_ref[...]
    t = jnp.dot(edges_ref[...], wc_ref[...], preferred_element_type=jnp.float32)
    selfp = jnp.dot(n, wa_ref[...], preferred_element_type=jnp.float32)
    s3 = jnp.broadcast_to(selfp[:, None, :], (p, nv, nl)).reshape(p * nv, nl)
    h = jnp.maximum(t + gath_ref[...] + s3 + be1_ref[...], 0.0)
    ne = jnp.dot(h, we2_ref[...], preferred_element_type=jnp.float32) + be2_ref[...]
    ne_ref[...] = ne
    agg = jnp.sum(ne.reshape(p, nv, nl), axis=1)
    h2 = jnp.maximum(
        jnp.dot(n, wn1a_ref[...], preferred_element_type=jnp.float32)
        + jnp.dot(agg, wn1b_ref[...], preferred_element_type=jnp.float32)
        + bn1_ref[...], 0.0)
    nn_ref[...] = n + jnp.dot(h2, wn2_ref[...],
                              preferred_element_type=jnp.float32) + bn2_ref[...]


def _edge_node(nodes2d, edges2d, gathered, wa, wc, we2, wn1a, wn1b, wn2,
               be1, be2, bn1, bn2, blk=1000, interpret=False):
    n_p, nl = nodes2d.shape
    e = edges2d.shape[0]
    nv = e // n_p
    grid = (n_p // blk,)
    wspec = pl.BlockSpec((nl, nl), lambda i: (0, 0))
    bspec = pl.BlockSpec((1, nl), lambda i: (0, 0))
    return pl.pallas_call(
        _edge_node_body,
        grid=grid,
        in_specs=[
            pl.BlockSpec((blk, nl), lambda i: (i, 0)),
            pl.BlockSpec((blk * nv, nl), lambda i: (i, 0)),
            pl.BlockSpec((blk * nv, nl), lambda i: (i, 0)),
            wspec, wspec, wspec, wspec, wspec, wspec,
            bspec, bspec, bspec, bspec,
        ],
        out_specs=[
            pl.BlockSpec((blk, nl), lambda i: (i, 0)),
            pl.BlockSpec((blk * nv, nl), lambda i: (i, 0)),
        ],
        out_shape=[
            jax.ShapeDtypeStruct((n_p, nl), jnp.float32),
            jax.ShapeDtypeStruct((e, nl), jnp.float32),
        ],
        interpret=interpret,
    )(nodes2d, edges2d, gathered, wa, wc, we2, wn1a, wn1b, wn2,
      be1, be2, bn1, bn2)


def kernel(nodes, edges_attributes, edges_indices, We1, be1, We2, be2,
           Wn1, bn1, Wn2, bn2):
    b, n_p, nl = nodes.shape
    nv = edges_attributes.shape[1]
    e = n_p * nv

    nodes2d = nodes.reshape(n_p, nl)
    edges2d = edges_attributes.reshape(e, nl)
    wa, wb, wc = We1[:nl], We1[nl:2 * nl], We1[2 * nl:]
    wn1a, wn1b = Wn1[:nl], Wn1[nl:]
    be1_2 = be1.reshape(1, nl)
    be2_2 = be2.reshape(1, nl)
    bn1_2 = bn1.reshape(1, nl)
    bn2_2 = bn2.reshape(1, nl)

    neighproj = _project(nodes2d, wb)

    ch = 128
    chunk = _NW * ch
    e_pad = ((e + chunk - 1) // chunk) * chunk
    flat_idx = edges_indices.reshape(e).astype(jnp.int32)
    flat_idx = jnp.pad(flat_idx, (0, e_pad - e))
    idx2d = flat_idx.reshape(e_pad // ch, ch)

    gathered = _sc_gather(neighproj, idx2d)

    new_nodes2d, new_edges2d = _edge_node(
        nodes2d, edges2d, gathered, wa, wc, We2, wn1a, wn1b, Wn2,
        be1_2, be2_2, bn1_2, bn2_2)

    return (new_nodes2d.reshape(b, n_p, nl),
            new_edges2d.reshape(n_p, nv, nl))
